# baseline (speedup 1.0000x reference)
import functools

import jax
import jax.numpy as jnp
import numpy as np
from jax.experimental import pallas as pl

_N = 5064
_NP = 5120
_NGT = 64
_NFGS = 128
_NBGS = 384
_NS = 512
_BIG = 1 << 20
_CHUNK = 64


def _priority_order():
    pr = np.asarray(jax.random.uniform(jax.random.key(42), (_N,)))
    return np.argsort(-pr, kind="stable").astype(np.int32)


_ORDER = _priority_order()


def _cumsum_lanes(x):
    d = 1
    while d < _NP:
        shifted = jnp.concatenate(
            [jnp.zeros((1, d), x.dtype), x[:, : _NP - d]], axis=1
        )
        x = x + shifted
        d *= 2
    return x


def _match(pt, gtb, gtc, valid):
    px1, py1, px2, py2 = (pt[i : i + 1, :] for i in range(4))
    gx1, gy1, gx2, gy2 = (gtb[:, i : i + 1] for i in range(4))
    area_p = (px2 - px1) * (py2 - py1)
    area_g = (gx2 - gx1) * (gy2 - gy1)
    ltx = jnp.maximum(gx1, px1)
    lty = jnp.maximum(gy1, py1)
    rbx = jnp.minimum(gx2, px2)
    rby = jnp.minimum(gy2, py2)
    w = jnp.maximum(rbx - ltx, 0.0)
    h = jnp.maximum(rby - lty, 0.0)
    inter = w * h
    union = area_g + area_p - inter
    iou = jnp.where(inter > 0, inter / jnp.maximum(union, 1e-9), 0.0)
    vals = jnp.max(iou, axis=0, keepdims=True)
    iota_g = jax.lax.broadcasted_iota(jnp.int32, (_NGT, _NP), 0)
    midx = jnp.min(jnp.where(iou == vals, iota_g, _NGT), axis=0, keepdims=True)
    fg = (vals >= 0.5) & valid
    onehot = (midx == iota_g).astype(jnp.int32)
    cls = jnp.sum(onehot * gtc, axis=0, keepdims=True)
    cls = jnp.where(fg, cls, 80)
    return vals, midx, fg, cls.astype(jnp.float32)


def _body(pt_a_ref, pt_p_ref, gtb_ref, gtc_ref, perm_ref, o1_ref, o2_ref):
    gtb = gtb_ref[...]
    gtc = gtc_ref[...]
    iota_k = jax.lax.broadcasted_iota(jnp.int32, (1, _NP), 1)
    valid = iota_k < _N

    vals_p, midx_p, fg_p, cls_p = _match(pt_p_ref[...], gtb, gtc, valid)
    vals_a, midx_a, fg_a, cls_a = _match(pt_a_ref[...], gtb, gtc, valid)

    cvalid = jnp.minimum(iota_k + 1, _N)
    fgm_p = fg_p
    bgm_p = (~fg_p) & valid
    cfg_p = _cumsum_lanes(fgm_p.astype(jnp.int32))
    cbg_p = cvalid - cfg_p
    slot_p = jnp.where(
        fgm_p & (cfg_p <= _NFGS),
        cfg_p - 1,
        jnp.where(bgm_p & (cbg_p <= _NBGS), _NFGS + cbg_p - 1, _BIG),
    )

    fgm_a = fg_a
    bgm_a = (~fg_a) & valid
    cfg_a = _cumsum_lanes(fgm_a.astype(jnp.int32))
    cbg_a = cvalid - cfg_a
    nfg = jnp.sum(fgm_a.astype(jnp.int32), axis=1, keepdims=True)
    nbg = _N - nfg
    fg_fill = nfg + cbg_a - 1
    bg_fill = _NFGS + nbg + cfg_a - 1
    slot_a = jnp.where(
        bgm_a & (fg_fill < _NFGS),
        fg_fill,
        jnp.where(fgm_a & (bg_fill < _NS), bg_fill, _BIG),
    )

    perm_f = perm_ref[...].astype(jnp.float32)
    iota_f = iota_k.astype(jnp.float32)
    v_p = jnp.concatenate(
        [perm_f, cls_p, vals_p, pt_p_ref[...], midx_p.astype(jnp.float32)], axis=0
    )
    v_a = jnp.concatenate(
        [iota_f, cls_a, vals_a, pt_a_ref[...], midx_a.astype(jnp.float32)], axis=0
    )

    gtb_f = gtb
    dn = (((1,), (1,)), ((), ()))
    for c in range(_NS // _CHUNK):
        iota_j = jax.lax.broadcasted_iota(jnp.int32, (_CHUNK, _NP), 0) + c * _CHUNK
        m1 = (slot_p == iota_j).astype(jnp.float32)
        m2 = (slot_a == iota_j).astype(jnp.float32)
        chunk = jax.lax.dot_general(
            m1, v_p, dn, precision=jax.lax.Precision.HIGHEST,
            preferred_element_type=jnp.float32,
        ) + jax.lax.dot_general(
            m2, v_a, dn, precision=jax.lax.Precision.HIGHEST,
            preferred_element_type=jnp.float32,
        )
        midx_c = jnp.round(chunk[:, 7:8]).astype(jnp.int32)
        ohg = (midx_c == jax.lax.broadcasted_iota(jnp.int32, (_CHUNK, _NGT), 1))
        gt_chunk = jnp.dot(
            ohg.astype(jnp.float32), gtb_f,
            precision=jax.lax.Precision.HIGHEST,
            preferred_element_type=jnp.float32,
        )
        o1_ref[c * _CHUNK : (c + 1) * _CHUNK, :] = chunk
        o2_ref[c * _CHUNK : (c + 1) * _CHUNK, :] = gt_chunk


def kernel(proposal_boxes, gt_boxes, gt_classes):
    order = _ORDER
    props = jnp.concatenate([proposal_boxes, gt_boxes], axis=0)
    pad = jnp.zeros((_NP - _N, 4), jnp.float32)
    pt_a = jnp.concatenate([props, pad], axis=0).T
    pt_p = jnp.concatenate([jnp.take(props, order, axis=0), pad], axis=0).T
    perm = jnp.concatenate([order, np.zeros((_NP - _N,), np.int32)]).reshape(1, _NP)
    gtc = gt_classes.astype(jnp.int32).reshape(_NGT, 1)

    o1, o2 = pl.pallas_call(
        _body,
        out_shape=[
            jax.ShapeDtypeStruct((_NS, 8), jnp.float32),
            jax.ShapeDtypeStruct((_NS, 4), jnp.float32),
        ],
    )(pt_a, pt_p, gt_boxes, gtc, perm)

    sampled_idxs = jnp.round(o1[:, 0]).astype(jnp.int32)
    sampled_classes = jnp.round(o1[:, 1]).astype(jnp.int32)
    sampled_ious = o1[:, 2]
    sampled_boxes = o1[:, 3:7]
    sampled_gt_boxes = o2
    return sampled_idxs, sampled_classes, sampled_ious, sampled_boxes, sampled_gt_boxes
